# Initial kernel scaffold; baseline (speedup 1.0000x reference)
import jax
import jax.numpy as jnp
from jax.experimental import pallas as pl


def kernel(x, edge_index, W1, b1, Wr, br, W2, b2):
    raise NotImplementedError("write your pallas kernel here")


# speedup vs baseline: 11.3754x; 11.3754x over previous
import jax
import jax.numpy as jnp
from jax import lax
from jax.experimental import pallas as pl
from jax.experimental.pallas import tpu as pltpu
from jax.experimental.pallas import tpu_sc as plsc

N = 10000
E = 320000
D = 128
HALF = 64
K_PROP = 10
TILES = 16
NPAD = 10240
NPT = NPAD // TILES
EPT = E // TILES
CHUNK = 80
NCHUNK = EPT // CHUNK
ROWB = 64
RB_ITERS = NPT // ROWB


def _mlp_body(x_ref, w1_ref, b1_ref, wr_ref, br_ref, w2_ref, b2_ref, o_ref):
    x = x_ref[...]
    h1 = jnp.maximum(
        jnp.dot(x, w1_ref[...], preferred_element_type=jnp.float32) + b1_ref[...], 0.0)
    h2 = jnp.maximum(
        jnp.dot(h1, wr_ref[...], preferred_element_type=jnp.float32) + br_ref[...], 0.0)
    o_ref[...] = jnp.dot(
        h1 + h2, w2_ref[...], preferred_element_type=jnp.float32) + b2_ref[...]


def _mlp(x, W1, b1, Wr, br, W2, b2):
    BM = 2000
    full = lambda i: (0, 0)
    return pl.pallas_call(
        _mlp_body,
        grid=(N // BM,),
        in_specs=[
            pl.BlockSpec((BM, D), lambda i: (i, 0)),
            pl.BlockSpec((D, D), full),
            pl.BlockSpec((1, D), full),
            pl.BlockSpec((D, D), full),
            pl.BlockSpec((1, D), full),
            pl.BlockSpec((D, D), full),
            pl.BlockSpec((1, D), full),
        ],
        out_specs=pl.BlockSpec((BM, D), lambda i: (i, 0)),
        out_shape=jax.ShapeDtypeStruct((N, D), jnp.float32),
    )(x, W1, b1.reshape(1, D), Wr, br.reshape(1, D), W2, b2.reshape(1, D))


def _lane(vec, k):
    idx = jnp.full((16,), k, dtype=jnp.int32)
    return vec.at[idx].get(mode="promise_in_bounds")


def _prop_body(src_hbm, dst_hbm, h_hbm,
               res_hbm, ya_hbm, yb_hbm, g_hbm,
               agg_sp, deg_sp,
               asrc_v, adst_v, ones_v, degb_v, dis_v, a_v, a2_v, sqd_v,
               rows0_v, rows1_v, acc_v, gbuf_v, sem0, sem1):
    c = lax.axis_index("c")
    tid = lax.axis_index("s")
    node_base = tid * NPT
    half_base = c * NPAD

    pltpu.sync_copy(src_hbm.at[tid], asrc_v)
    pltpu.sync_copy(dst_hbm.at[tid], adst_v)

    cvec = jnp.full((16,), c * NPAD, dtype=jnp.int32)

    def _off(j, carry):
        for l in range(CHUNK // 16):
            asrc_v[j, pl.ds(l * 16, 16)] = asrc_v[j, pl.ds(l * 16, 16)] + cvec
        return carry
    lax.fori_loop(0, NCHUNK, _off, 0)

    ones16 = jnp.ones((16,), jnp.float32)
    zeros16 = jnp.zeros((16,), jnp.float32)
    for l in range(CHUNK // 16):
        ones_v[pl.ds(l * 16, 16)] = ones16

    def _zdeg(i, carry):
        degb_v[pl.ds(i * 16, 16)] = zeros16
        return carry
    lax.fori_loop(0, NPT // 16, _zdeg, 0)
    pltpu.sync_copy(degb_v, deg_sp.at[pl.ds(node_base, NPT)])
    plsc.subcore_barrier()

    def _deg(j, carry):
        pltpu.sync_copy(ones_v, deg_sp.at[adst_v.at[j]], add=True)
        return carry
    lax.fori_loop(0, NCHUNK, _deg, 0)
    plsc.subcore_barrier()

    pltpu.sync_copy(deg_sp.at[pl.ds(node_base, NPT)], degb_v)

    def _prep(i, carry):
        dloc = degb_v[pl.ds(i * 16, 16)] + 1.0
        sq = dloc
        for _ in range(16):
            sq = 0.5 * (sq + dloc / sq)
        r = 1.0 / sq
        a = 0.9 / dloc
        dis_v[pl.ds(i * 16, 16)] = r
        a_v[pl.ds(i * 16, 16)] = a
        a2_v[pl.ds(i * 16, 16)] = a * sq
        sqd_v[pl.ds(i * 16, 16)] = sq
        return carry
    lax.fori_loop(0, NPT // 16, _prep, 0)

    def _gy(ci, carry):
        rbase = node_base + ci * ROWB
        gb = half_base + rbase
        pltpu.sync_copy(h_hbm.at[pl.ds(gb, ROWB)], acc_v)

        def _grp(ng, carry2):
            s16v = dis_v[pl.ds(ci * ROWB + ng * 16, 16)]
            for k in range(16):
                s16 = _lane(s16v, k)
                nn = ng * 16 + k
                for q in range(HALF // 16):
                    y0 = acc_v[nn, pl.ds(q * 16, 16)] * s16
                    acc_v[nn, pl.ds(q * 16, 16)] = y0
                    gbuf_v[nn, pl.ds(q * 16, 16)] = y0 * 0.1
            return carry2
        lax.fori_loop(0, ROWB // 16, _grp, 0)
        pltpu.sync_copy(acc_v, ya_hbm.at[pl.ds(gb, ROWB)])
        pltpu.sync_copy(gbuf_v, g_hbm.at[pl.ds(gb, ROWB)])
        return carry
    lax.fori_loop(0, RB_ITERS, _gy, 0)
    plsc.subcore_barrier()

    ybufs = [ya_hbm, yb_hbm]
    for r in range(K_PROP):
        ycur = ybufs[r % 2]
        final = r == K_PROP - 1
        ynext = res_hbm if final else ybufs[(r + 1) % 2]

        def _init(ci, carry):
            rbase = node_base + ci * ROWB
            pltpu.sync_copy(ycur.at[pl.ds(half_base + rbase, ROWB)], acc_v)
            pltpu.sync_copy(acc_v, agg_sp.at[pl.ds(rbase, ROWB)])
            return carry
        lax.fori_loop(0, RB_ITERS, _init, 0)
        plsc.subcore_barrier()

        pltpu.async_copy(ycur.at[asrc_v.at[0]], rows0_v, sem0)

        def _edges(jj, carry):
            ja = 2 * jj
            jb = 2 * jj + 1
            pltpu.make_async_copy(ycur.at[asrc_v.at[ja]], rows0_v, sem0).wait()
            pltpu.async_copy(ycur.at[asrc_v.at[jb]], rows1_v, sem1)
            pltpu.sync_copy(rows0_v, agg_sp.at[adst_v.at[ja]], add=True)
            pltpu.make_async_copy(ycur.at[asrc_v.at[jb]], rows1_v, sem1).wait()

            @pl.when(jj < NCHUNK // 2 - 1)
            def _():
                pltpu.async_copy(ycur.at[asrc_v.at[ja + 2]], rows0_v, sem0)

            pltpu.sync_copy(rows1_v, agg_sp.at[adst_v.at[jb]], add=True)
            return carry
        lax.fori_loop(0, NCHUNK // 2, _edges, 0)
        plsc.subcore_barrier()

        av = a2_v if final else a_v

        def _upd(ci, carry):
            rbase = node_base + ci * ROWB
            gb = half_base + rbase
            pltpu.sync_copy(agg_sp.at[pl.ds(rbase, ROWB)], acc_v)
            pltpu.sync_copy(g_hbm.at[pl.ds(gb, ROWB)], gbuf_v)

            def _grp(ng, carry2):
                a16v = av[pl.ds(ci * ROWB + ng * 16, 16)]
                if final:
                    q16v = sqd_v[pl.ds(ci * ROWB + ng * 16, 16)]
                for k in range(16):
                    a16 = _lane(a16v, k)
                    if final:
                        q16 = _lane(q16v, k)
                    nn = ng * 16 + k
                    for q in range(HALF // 16):
                        t = acc_v[nn, pl.ds(q * 16, 16)] * a16
                        gg = gbuf_v[nn, pl.ds(q * 16, 16)]
                        if final:
                            gg = gg * q16
                        acc_v[nn, pl.ds(q * 16, 16)] = t + gg
                return carry2
            lax.fori_loop(0, ROWB // 16, _grp, 0)
            pltpu.sync_copy(acc_v, ynext.at[pl.ds(gb, ROWB)])
            return carry
        lax.fori_loop(0, RB_ITERS, _upd, 0)
        plsc.subcore_barrier()


def _propagate(src_r, dst_r, h_cols):
    mesh = plsc.VectorSubcoreMesh(core_axis_name="c", subcore_axis_name="s")
    f32 = jnp.float32
    out_type = [
        jax.ShapeDtypeStruct((2 * NPAD, HALF), f32),
        jax.ShapeDtypeStruct((2 * NPAD, HALF), f32),
        jax.ShapeDtypeStruct((2 * NPAD, HALF), f32),
        jax.ShapeDtypeStruct((2 * NPAD, HALF), f32),
    ]
    scratch_types = [
        pltpu.VMEM_SHARED((NPAD, HALF), f32),
        pltpu.VMEM_SHARED((NPAD,), f32),
        pltpu.VMEM((NCHUNK, CHUNK), jnp.int32),
        pltpu.VMEM((NCHUNK, CHUNK), jnp.int32),
        pltpu.VMEM((CHUNK,), f32),
        pltpu.VMEM((NPT,), f32),
        pltpu.VMEM((NPT,), f32),
        pltpu.VMEM((NPT,), f32),
        pltpu.VMEM((NPT,), f32),
        pltpu.VMEM((NPT,), f32),
        pltpu.VMEM((CHUNK, HALF), f32),
        pltpu.VMEM((CHUNK, HALF), f32),
        pltpu.VMEM((ROWB, HALF), f32),
        pltpu.VMEM((ROWB, HALF), f32),
        pltpu.SemaphoreType.DMA,
        pltpu.SemaphoreType.DMA,
    ]
    res, _, _, _ = pl.kernel(
        _prop_body,
        out_type=out_type,
        mesh=mesh,
        scratch_types=scratch_types,
        compiler_params=pltpu.CompilerParams(use_tc_tiling_on_sc=False),
    )(src_r, dst_r, h_cols)
    return res


def kernel(x, edge_index, W1, b1, Wr, br, W2, b2):
    h = _mlp(x, W1, b1, Wr, br, W2, b2)
    hp = jnp.pad(h, ((0, NPAD - N), (0, 0)))
    h_cols = jnp.concatenate([hp[:, :HALF], hp[:, HALF:]], axis=0)
    src_r = edge_index[0].reshape(TILES, NCHUNK, CHUNK)
    dst_r = edge_index[1].reshape(TILES, NCHUNK, CHUNK)
    res = _propagate(src_r, dst_r, h_cols)
    return jnp.concatenate([res[:N], res[NPAD:NPAD + N]], axis=1)


# speedup vs baseline: 14.3629x; 1.2626x over previous
import jax
import jax.numpy as jnp
from jax import lax
from jax.experimental import pallas as pl
from jax.experimental.pallas import tpu as pltpu
from jax.experimental.pallas import tpu_sc as plsc

N = 10000
E = 320000
D = 128
HALF = 64
K_PROP = 10
TILES = 16
NPAD = 10240
NPT = NPAD // TILES
CHUNK = 128
NCHUNK = 160
EPAD = TILES * NCHUNK * CHUNK
ROWB = 64
RB_ITERS = NPT // ROWB


def _mlp_body(x_ref, w1_ref, b1_ref, wr_ref, br_ref, w2_ref, b2_ref, o_ref):
    x = x_ref[...]
    h1 = jnp.maximum(
        jnp.dot(x, w1_ref[...], preferred_element_type=jnp.float32) + b1_ref[...], 0.0)
    h2 = jnp.maximum(
        jnp.dot(h1, wr_ref[...], preferred_element_type=jnp.float32) + br_ref[...], 0.0)
    o_ref[...] = jnp.dot(
        h1 + h2, w2_ref[...], preferred_element_type=jnp.float32) + b2_ref[...]


def _mlp(x, W1, b1, Wr, br, W2, b2):
    BM = 2000
    full = lambda i: (0, 0)
    return pl.pallas_call(
        _mlp_body,
        grid=(N // BM,),
        in_specs=[
            pl.BlockSpec((BM, D), lambda i: (i, 0)),
            pl.BlockSpec((D, D), full),
            pl.BlockSpec((1, D), full),
            pl.BlockSpec((D, D), full),
            pl.BlockSpec((1, D), full),
            pl.BlockSpec((D, D), full),
            pl.BlockSpec((1, D), full),
        ],
        out_specs=pl.BlockSpec((BM, D), lambda i: (i, 0)),
        out_shape=jax.ShapeDtypeStruct((N, D), jnp.float32),
    )(x, W1, b1.reshape(1, D), Wr, br.reshape(1, D), W2, b2.reshape(1, D))


def _lane(vec, k):
    idx = jnp.full((16,), k, dtype=jnp.int32)
    return vec.at[idx].get(mode="promise_in_bounds")


def _prop_body(src_hbm, dst_hbm, h_hbm,
               res_hbm, ya_hbm, yb_hbm, g_hbm,
               agg_sp, deg_sp,
               asrc_v, adst_v, ones_v, degb_v, dis_v, a_v, a2_v, sqd_v,
               rows0_v, rows1_v, acc_v, gbuf_v, sem0, sem1):
    c = lax.axis_index("c")
    tid = lax.axis_index("s")
    node_base = tid * NPT
    half_base = c * NPAD

    pltpu.sync_copy(src_hbm.at[tid], asrc_v)
    pltpu.sync_copy(dst_hbm.at[tid], adst_v)

    cvec = jnp.full((16,), c * NPAD, dtype=jnp.int32)

    def _off(j, carry):
        for l in range(CHUNK // 16):
            asrc_v[j, pl.ds(l * 16, 16)] = asrc_v[j, pl.ds(l * 16, 16)] + cvec
        return carry
    lax.fori_loop(0, NCHUNK, _off, 0)

    ones16 = jnp.ones((16,), jnp.float32)
    zeros16 = jnp.zeros((16,), jnp.float32)
    for l in range(CHUNK // 16):
        ones_v[pl.ds(l * 16, 16)] = ones16

    def _zdeg(i, carry):
        degb_v[pl.ds(i * 16, 16)] = zeros16
        return carry
    lax.fori_loop(0, NPT // 16, _zdeg, 0)
    pltpu.sync_copy(degb_v, deg_sp.at[pl.ds(node_base, NPT)])
    plsc.subcore_barrier()

    def _deg(j, carry):
        pltpu.sync_copy(ones_v, deg_sp.at[adst_v.at[j]], add=True)
        return carry
    lax.fori_loop(0, NCHUNK, _deg, 0)
    plsc.subcore_barrier()

    pltpu.sync_copy(deg_sp.at[pl.ds(node_base, NPT)], degb_v)

    def _prep(i, carry):
        dloc = degb_v[pl.ds(i * 16, 16)] + 1.0
        sq = dloc
        for _ in range(16):
            sq = 0.5 * (sq + dloc / sq)
        r = 1.0 / sq
        a = 0.9 / dloc
        dis_v[pl.ds(i * 16, 16)] = r
        a_v[pl.ds(i * 16, 16)] = a
        a2_v[pl.ds(i * 16, 16)] = a * sq
        sqd_v[pl.ds(i * 16, 16)] = sq
        return carry
    lax.fori_loop(0, NPT // 16, _prep, 0)

    def _gy(ci, carry):
        rbase = node_base + ci * ROWB
        gb = half_base + rbase
        pltpu.sync_copy(h_hbm.at[pl.ds(gb, ROWB)], acc_v)

        def _grp(ng, carry2):
            s16v = dis_v[pl.ds(ci * ROWB + ng * 16, 16)]
            for k in range(16):
                s16 = _lane(s16v, k)
                nn = ng * 16 + k
                for q in range(HALF // 16):
                    y0 = acc_v[nn, pl.ds(q * 16, 16)] * s16
                    acc_v[nn, pl.ds(q * 16, 16)] = y0
                    gbuf_v[nn, pl.ds(q * 16, 16)] = y0 * 0.1
            return carry2
        lax.fori_loop(0, ROWB // 16, _grp, 0)
        pltpu.sync_copy(acc_v, ya_hbm.at[pl.ds(gb, ROWB)])
        pltpu.sync_copy(gbuf_v, g_hbm.at[pl.ds(gb, ROWB)])
        return carry
    lax.fori_loop(0, RB_ITERS, _gy, 0)
    plsc.subcore_barrier()

    ybufs = [ya_hbm, yb_hbm]
    for r in range(K_PROP):
        ycur = ybufs[r % 2]
        final = r == K_PROP - 1
        ynext = res_hbm if final else ybufs[(r + 1) % 2]

        def _init(ci, carry):
            rbase = node_base + ci * ROWB
            pltpu.sync_copy(ycur.at[pl.ds(half_base + rbase, ROWB)], acc_v)
            pltpu.sync_copy(acc_v, agg_sp.at[pl.ds(rbase, ROWB)])
            return carry
        lax.fori_loop(0, RB_ITERS, _init, 0)
        plsc.subcore_barrier()

        pltpu.async_copy(ycur.at[asrc_v.at[0]], rows0_v, sem0)

        def _edges(jj, carry):
            ja = 2 * jj
            jb = 2 * jj + 1
            pltpu.make_async_copy(ycur.at[asrc_v.at[ja]], rows0_v, sem0).wait()
            pltpu.async_copy(ycur.at[asrc_v.at[jb]], rows1_v, sem1)
            pltpu.sync_copy(rows0_v, agg_sp.at[adst_v.at[ja]], add=True)
            pltpu.make_async_copy(ycur.at[asrc_v.at[jb]], rows1_v, sem1).wait()

            @pl.when(jj < NCHUNK // 2 - 1)
            def _():
                pltpu.async_copy(ycur.at[asrc_v.at[ja + 2]], rows0_v, sem0)

            pltpu.sync_copy(rows1_v, agg_sp.at[adst_v.at[jb]], add=True)
            return carry
        lax.fori_loop(0, NCHUNK // 2, _edges, 0)
        plsc.subcore_barrier()

        av = a2_v if final else a_v

        def _upd(ci, carry):
            rbase = node_base + ci * ROWB
            gb = half_base + rbase
            pltpu.sync_copy(agg_sp.at[pl.ds(rbase, ROWB)], acc_v)
            pltpu.sync_copy(g_hbm.at[pl.ds(gb, ROWB)], gbuf_v)

            def _grp(ng, carry2):
                a16v = av[pl.ds(ci * ROWB + ng * 16, 16)]
                if final:
                    q16v = sqd_v[pl.ds(ci * ROWB + ng * 16, 16)]
                for k in range(16):
                    a16 = _lane(a16v, k)
                    if final:
                        q16 = _lane(q16v, k)
                    nn = ng * 16 + k
                    for q in range(HALF // 16):
                        t = acc_v[nn, pl.ds(q * 16, 16)] * a16
                        gg = gbuf_v[nn, pl.ds(q * 16, 16)]
                        if final:
                            gg = gg * q16
                        acc_v[nn, pl.ds(q * 16, 16)] = t + gg
                return carry2
            lax.fori_loop(0, ROWB // 16, _grp, 0)
            pltpu.sync_copy(acc_v, ynext.at[pl.ds(gb, ROWB)])
            return carry
        lax.fori_loop(0, RB_ITERS, _upd, 0)
        plsc.subcore_barrier()


def _propagate(src_r, dst_r, h_cols):
    mesh = plsc.VectorSubcoreMesh(core_axis_name="c", subcore_axis_name="s")
    f32 = jnp.float32
    out_type = [
        jax.ShapeDtypeStruct((2 * NPAD, HALF), f32),
        jax.ShapeDtypeStruct((2 * NPAD, HALF), f32),
        jax.ShapeDtypeStruct((2 * NPAD, HALF), f32),
        jax.ShapeDtypeStruct((2 * NPAD, HALF), f32),
    ]
    scratch_types = [
        pltpu.VMEM_SHARED((NPAD, HALF), f32),
        pltpu.VMEM_SHARED((NPAD,), f32),
        pltpu.VMEM((NCHUNK, CHUNK), jnp.int32),
        pltpu.VMEM((NCHUNK, CHUNK), jnp.int32),
        pltpu.VMEM((CHUNK,), f32),
        pltpu.VMEM((NPT,), f32),
        pltpu.VMEM((NPT,), f32),
        pltpu.VMEM((NPT,), f32),
        pltpu.VMEM((NPT,), f32),
        pltpu.VMEM((NPT,), f32),
        pltpu.VMEM((CHUNK, HALF), f32),
        pltpu.VMEM((CHUNK, HALF), f32),
        pltpu.VMEM((ROWB, HALF), f32),
        pltpu.VMEM((ROWB, HALF), f32),
        pltpu.SemaphoreType.DMA,
        pltpu.SemaphoreType.DMA,
    ]
    res, _, _, _ = pl.kernel(
        _prop_body,
        out_type=out_type,
        mesh=mesh,
        scratch_types=scratch_types,
        compiler_params=pltpu.CompilerParams(use_tc_tiling_on_sc=False),
    )(src_r, dst_r, h_cols)
    return res


def kernel(x, edge_index, W1, b1, Wr, br, W2, b2):
    h = _mlp(x, W1, b1, Wr, br, W2, b2)
    hp = jnp.pad(h, ((0, NPAD - N), (0, 0)))
    h_cols = jnp.concatenate([hp[:, :HALF], hp[:, HALF:]], axis=0)
    padidx = N + (jnp.arange(EPAD - E, dtype=jnp.int32) % (NPAD - N))
    src_r = jnp.concatenate([edge_index[0], padidx]).reshape(TILES, NCHUNK, CHUNK)
    dst_r = jnp.concatenate([edge_index[1], padidx]).reshape(TILES, NCHUNK, CHUNK)
    res = _propagate(src_r, dst_r, h_cols)
    return jnp.concatenate([res[:N], res[NPAD:NPAD + N]], axis=1)


# speedup vs baseline: 20.3702x; 1.4183x over previous
import jax
import jax.numpy as jnp
from jax import lax
from jax.experimental import pallas as pl
from jax.experimental.pallas import tpu as pltpu
from jax.experimental.pallas import tpu_sc as plsc

N = 10000
E = 320000
D = 128
HALF = 64
K_PROP = 10
TILES = 16
NPAD = 10240
NPT = NPAD // TILES
CHUNK = 128
NCHUNK = 160
NSLOT = 4
NJ = NCHUNK // NSLOT
EPAD = TILES * NCHUNK * CHUNK
ROWB = 64
RB_ITERS = NPT // ROWB


def _mlp_body(x_ref, w1_ref, b1_ref, wr_ref, br_ref, w2_ref, b2_ref, o_ref):
    x = x_ref[...]
    h1 = jnp.maximum(
        jnp.dot(x, w1_ref[...], preferred_element_type=jnp.float32) + b1_ref[...], 0.0)
    h2 = jnp.maximum(
        jnp.dot(h1, wr_ref[...], preferred_element_type=jnp.float32) + br_ref[...], 0.0)
    o_ref[...] = jnp.dot(
        h1 + h2, w2_ref[...], preferred_element_type=jnp.float32) + b2_ref[...]


def _mlp(x, W1, b1, Wr, br, W2, b2):
    BM = 2000
    full = lambda i: (0, 0)
    return pl.pallas_call(
        _mlp_body,
        grid=(N // BM,),
        in_specs=[
            pl.BlockSpec((BM, D), lambda i: (i, 0)),
            pl.BlockSpec((D, D), full),
            pl.BlockSpec((1, D), full),
            pl.BlockSpec((D, D), full),
            pl.BlockSpec((1, D), full),
            pl.BlockSpec((D, D), full),
            pl.BlockSpec((1, D), full),
        ],
        out_specs=pl.BlockSpec((BM, D), lambda i: (i, 0)),
        out_shape=jax.ShapeDtypeStruct((N, D), jnp.float32),
    )(x, W1, b1.reshape(1, D), Wr, br.reshape(1, D), W2, b2.reshape(1, D))


def _lane(vec, k):
    idx = jnp.full((16,), k, dtype=jnp.int32)
    return vec.at[idx].get(mode="promise_in_bounds")


def _prop_body(src_hbm, dst_hbm, h_hbm,
               res_hbm, ya_hbm, yb_hbm, g_hbm,
               agg_sp, deg_sp,
               asrc_v, adst_v, ones_v, degb_v, dis_v, a_v, a2_v, sqd_v,
               rows_v0, rows_v1, rows_v2, rows_v3, acc_v, gbuf_v,
               sg0, sg1, sg2, sg3, ss0, ss1, ss2, ss3):
    rows_v = [rows_v0, rows_v1, rows_v2, rows_v3]
    sg = [sg0, sg1, sg2, sg3]
    ss = [ss0, ss1, ss2, ss3]
    c = lax.axis_index("c")
    tid = lax.axis_index("s")
    node_base = tid * NPT
    half_base = c * NPAD

    pltpu.sync_copy(src_hbm.at[tid], asrc_v)
    pltpu.sync_copy(dst_hbm.at[tid], adst_v)

    cvec = jnp.full((16,), c * NPAD, dtype=jnp.int32)

    def _off(j, carry):
        for l in range(CHUNK // 16):
            asrc_v[j, pl.ds(l * 16, 16)] = asrc_v[j, pl.ds(l * 16, 16)] + cvec
        return carry
    lax.fori_loop(0, NCHUNK, _off, 0)

    ones16 = jnp.ones((16,), jnp.float32)
    zeros16 = jnp.zeros((16,), jnp.float32)
    for l in range(CHUNK // 16):
        ones_v[pl.ds(l * 16, 16)] = ones16

    def _zdeg(i, carry):
        degb_v[pl.ds(i * 16, 16)] = zeros16
        return carry
    lax.fori_loop(0, NPT // 16, _zdeg, 0)
    pltpu.sync_copy(degb_v, deg_sp.at[pl.ds(node_base, NPT)])
    plsc.subcore_barrier()

    def _deg(j, carry):
        pltpu.sync_copy(ones_v, deg_sp.at[adst_v.at[j]], add=True)
        return carry
    lax.fori_loop(0, NCHUNK, _deg, 0)
    plsc.subcore_barrier()

    pltpu.sync_copy(deg_sp.at[pl.ds(node_base, NPT)], degb_v)

    def _prep(i, carry):
        dloc = degb_v[pl.ds(i * 16, 16)] + 1.0
        sq = dloc
        for _ in range(16):
            sq = 0.5 * (sq + dloc / sq)
        r = 1.0 / sq
        a = 0.9 / dloc
        dis_v[pl.ds(i * 16, 16)] = r
        a_v[pl.ds(i * 16, 16)] = a
        a2_v[pl.ds(i * 16, 16)] = a * sq
        sqd_v[pl.ds(i * 16, 16)] = sq
        return carry
    lax.fori_loop(0, NPT // 16, _prep, 0)

    def _gy(ci, carry):
        rbase = node_base + ci * ROWB
        gb = half_base + rbase
        pltpu.sync_copy(h_hbm.at[pl.ds(gb, ROWB)], acc_v)

        def _grp(ng, carry2):
            s16v = dis_v[pl.ds(ci * ROWB + ng * 16, 16)]
            for k in range(16):
                s16 = _lane(s16v, k)
                nn = ng * 16 + k
                for q in range(HALF // 16):
                    y0 = acc_v[nn, pl.ds(q * 16, 16)] * s16
                    acc_v[nn, pl.ds(q * 16, 16)] = y0
                    gbuf_v[nn, pl.ds(q * 16, 16)] = y0 * 0.1
            return carry2
        lax.fori_loop(0, ROWB // 16, _grp, 0)
        pltpu.sync_copy(acc_v, ya_hbm.at[pl.ds(gb, ROWB)])
        pltpu.sync_copy(gbuf_v, g_hbm.at[pl.ds(gb, ROWB)])
        return carry
    lax.fori_loop(0, RB_ITERS, _gy, 0)
    plsc.subcore_barrier()

    ybufs = [ya_hbm, yb_hbm]
    for r in range(K_PROP):
        ycur = ybufs[r % 2]
        final = r == K_PROP - 1
        ynext = res_hbm if final else ybufs[(r + 1) % 2]

        def _init(ci, carry):
            rbase = node_base + ci * ROWB
            pltpu.sync_copy(ycur.at[pl.ds(half_base + rbase, ROWB)], acc_v)
            pltpu.sync_copy(acc_v, agg_sp.at[pl.ds(rbase, ROWB)])
            return carry
        lax.fori_loop(0, RB_ITERS, _init, 0)
        plsc.subcore_barrier()

        for s in range(NSLOT):
            pltpu.async_copy(ycur.at[asrc_v.at[s]], rows_v[s], sg[s])

        def _edges(jj, carry):
            for s in range(NSLOT):
                cidx = jj * NSLOT + s
                pltpu.make_async_copy(
                    ycur.at[asrc_v.at[cidx]], rows_v[s], sg[s]).wait()
                pltpu.async_copy(
                    rows_v[s], agg_sp.at[adst_v.at[cidx]], ss[s], add=True)
            for s in range(NSLOT):
                cidx = jj * NSLOT + s

                @pl.when(jj < NJ - 1)
                def _():
                    pltpu.make_async_copy(
                        rows_v[s], agg_sp.at[adst_v.at[cidx]], ss[s]).wait()
                    pltpu.async_copy(
                        ycur.at[asrc_v.at[cidx + NSLOT]], rows_v[s], sg[s])
            return carry
        lax.fori_loop(0, NJ, _edges, 0)
        for s in range(NSLOT):
            cidx = (NJ - 1) * NSLOT + s
            pltpu.make_async_copy(
                rows_v[s], agg_sp.at[adst_v.at[cidx]], ss[s]).wait()
        plsc.subcore_barrier()

        av = a2_v if final else a_v

        def _upd(ci, carry):
            rbase = node_base + ci * ROWB
            gb = half_base + rbase
            pltpu.sync_copy(agg_sp.at[pl.ds(rbase, ROWB)], acc_v)
            pltpu.sync_copy(g_hbm.at[pl.ds(gb, ROWB)], gbuf_v)

            def _grp(ng, carry2):
                a16v = av[pl.ds(ci * ROWB + ng * 16, 16)]
                if final:
                    q16v = sqd_v[pl.ds(ci * ROWB + ng * 16, 16)]
                for k in range(16):
                    a16 = _lane(a16v, k)
                    if final:
                        q16 = _lane(q16v, k)
                    nn = ng * 16 + k
                    for q in range(HALF // 16):
                        t = acc_v[nn, pl.ds(q * 16, 16)] * a16
                        gg = gbuf_v[nn, pl.ds(q * 16, 16)]
                        if final:
                            gg = gg * q16
                        acc_v[nn, pl.ds(q * 16, 16)] = t + gg
                return carry2
            lax.fori_loop(0, ROWB // 16, _grp, 0)
            pltpu.sync_copy(acc_v, ynext.at[pl.ds(gb, ROWB)])
            return carry
        lax.fori_loop(0, RB_ITERS, _upd, 0)
        plsc.subcore_barrier()


def _propagate(src_r, dst_r, h_cols):
    mesh = plsc.VectorSubcoreMesh(core_axis_name="c", subcore_axis_name="s")
    f32 = jnp.float32
    out_type = [
        jax.ShapeDtypeStruct((2 * NPAD, HALF), f32),
        jax.ShapeDtypeStruct((2 * NPAD, HALF), f32),
        jax.ShapeDtypeStruct((2 * NPAD, HALF), f32),
        jax.ShapeDtypeStruct((2 * NPAD, HALF), f32),
    ]
    scratch_types = [
        pltpu.VMEM_SHARED((NPAD, HALF), f32),
        pltpu.VMEM_SHARED((NPAD,), f32),
        pltpu.VMEM((NCHUNK, CHUNK), jnp.int32),
        pltpu.VMEM((NCHUNK, CHUNK), jnp.int32),
        pltpu.VMEM((CHUNK,), f32),
        pltpu.VMEM((NPT,), f32),
        pltpu.VMEM((NPT,), f32),
        pltpu.VMEM((NPT,), f32),
        pltpu.VMEM((NPT,), f32),
        pltpu.VMEM((NPT,), f32),
        pltpu.VMEM((CHUNK, HALF), f32),
        pltpu.VMEM((CHUNK, HALF), f32),
        pltpu.VMEM((CHUNK, HALF), f32),
        pltpu.VMEM((CHUNK, HALF), f32),
        pltpu.VMEM((ROWB, HALF), f32),
        pltpu.VMEM((ROWB, HALF), f32),
        pltpu.SemaphoreType.DMA,
        pltpu.SemaphoreType.DMA,
        pltpu.SemaphoreType.DMA,
        pltpu.SemaphoreType.DMA,
        pltpu.SemaphoreType.DMA,
        pltpu.SemaphoreType.DMA,
        pltpu.SemaphoreType.DMA,
        pltpu.SemaphoreType.DMA,
    ]
    res, _, _, _ = pl.kernel(
        _prop_body,
        out_type=out_type,
        mesh=mesh,
        scratch_types=scratch_types,
        compiler_params=pltpu.CompilerParams(use_tc_tiling_on_sc=False),
    )(src_r, dst_r, h_cols)
    return res


def kernel(x, edge_index, W1, b1, Wr, br, W2, b2):
    h = _mlp(x, W1, b1, Wr, br, W2, b2)
    hp = jnp.pad(h, ((0, NPAD - N), (0, 0)))
    h_cols = jnp.concatenate([hp[:, :HALF], hp[:, HALF:]], axis=0)
    padidx = N + (jnp.arange(EPAD - E, dtype=jnp.int32) % (NPAD - N))
    src_r = jnp.concatenate([edge_index[0], padidx]).reshape(TILES, NCHUNK, CHUNK)
    dst_r = jnp.concatenate([edge_index[1], padidx]).reshape(TILES, NCHUNK, CHUNK)
    res = _propagate(src_r, dst_r, h_cols)
    return jnp.concatenate([res[:N], res[NPAD:NPAD + N]], axis=1)


# speedup vs baseline: 21.6133x; 1.0610x over previous
import jax
import jax.numpy as jnp
from jax import lax
from jax.experimental import pallas as pl
from jax.experimental.pallas import tpu as pltpu
from jax.experimental.pallas import tpu_sc as plsc

N = 10000
E = 320000
D = 128
HALF = 64
K_PROP = 10
TILES = 16
NPAD = 10240
NPT = NPAD // TILES
CHUNK = 128
NCHUNK = 160
NSLOT = 4
NJ = NCHUNK // NSLOT
EPAD = TILES * NCHUNK * CHUNK
ROWB = 64
RB_ITERS = NPT // ROWB


def _mlp_body(x_ref, w1_ref, b1_ref, wr_ref, br_ref, w2_ref, b2_ref, o_ref):
    x = x_ref[...]
    h1 = jnp.maximum(
        jnp.dot(x, w1_ref[...], preferred_element_type=jnp.float32) + b1_ref[...], 0.0)
    h2 = jnp.maximum(
        jnp.dot(h1, wr_ref[...], preferred_element_type=jnp.float32) + br_ref[...], 0.0)
    o_ref[...] = jnp.dot(
        h1 + h2, w2_ref[...], preferred_element_type=jnp.float32) + b2_ref[...]


def _mlp(x, W1, b1, Wr, br, W2, b2):
    BM = 2000
    full = lambda i: (0, 0)
    return pl.pallas_call(
        _mlp_body,
        grid=(N // BM,),
        in_specs=[
            pl.BlockSpec((BM, D), lambda i: (i, 0)),
            pl.BlockSpec((D, D), full),
            pl.BlockSpec((1, D), full),
            pl.BlockSpec((D, D), full),
            pl.BlockSpec((1, D), full),
            pl.BlockSpec((D, D), full),
            pl.BlockSpec((1, D), full),
        ],
        out_specs=pl.BlockSpec((BM, D), lambda i: (i, 0)),
        out_shape=jax.ShapeDtypeStruct((N, D), jnp.float32),
    )(x, W1, b1.reshape(1, D), Wr, br.reshape(1, D), W2, b2.reshape(1, D))


def _lane(vec, k):
    idx = jnp.full((16,), k, dtype=jnp.int32)
    return vec.at[idx].get(mode="promise_in_bounds")


def _prop_body(src_hbm, dst_hbm, h_hbm,
               res_hbm, ya_hbm, yb_hbm, g_hbm,
               agg_sp, deg_sp,
               asrc_v, adst_v, ones_v, degb_v, dis_v, a_v, a2_v, sqd_v,
               rows_v0, rows_v1, rows_v2, rows_v3, acc_v, gbuf_v,
               sg0, sg1, sg2, sg3, ss0, ss1, ss2, ss3):
    rows_v = [rows_v0, rows_v1, rows_v2, rows_v3]
    sg = [sg0, sg1, sg2, sg3]
    ss = [ss0, ss1, ss2, ss3]
    c = lax.axis_index("c")
    tid = lax.axis_index("s")
    node_base = tid * NPT
    half_base = c * NPAD

    pltpu.sync_copy(src_hbm.at[tid], asrc_v)
    pltpu.sync_copy(dst_hbm.at[tid], adst_v)

    cvec = jnp.full((16,), c * NPAD, dtype=jnp.int32)

    def _off(j, carry):
        for l in range(CHUNK // 16):
            asrc_v[j, pl.ds(l * 16, 16)] = asrc_v[j, pl.ds(l * 16, 16)] + cvec
        return carry
    lax.fori_loop(0, NCHUNK, _off, 0)

    ones16 = jnp.ones((16,), jnp.float32)
    zeros16 = jnp.zeros((16,), jnp.float32)
    for l in range(CHUNK // 16):
        ones_v[pl.ds(l * 16, 16)] = ones16

    def _zdeg(i, carry):
        degb_v[pl.ds(i * 16, 16)] = zeros16
        return carry
    lax.fori_loop(0, NPT // 16, _zdeg, 0)
    pltpu.sync_copy(degb_v, deg_sp.at[pl.ds(node_base, NPT)])
    plsc.subcore_barrier()

    def _deg(j, carry):
        pltpu.sync_copy(ones_v, deg_sp.at[adst_v.at[j]], add=True)
        return carry
    lax.fori_loop(0, NCHUNK, _deg, 0)
    plsc.subcore_barrier()

    pltpu.sync_copy(deg_sp.at[pl.ds(node_base, NPT)], degb_v)

    def _prep(i, carry):
        dloc = degb_v[pl.ds(i * 16, 16)] + 1.0
        sq = dloc
        for _ in range(16):
            sq = 0.5 * (sq + dloc / sq)
        r = 1.0 / sq
        a = 0.9 / dloc
        dis_v[pl.ds(i * 16, 16)] = r
        a_v[pl.ds(i * 16, 16)] = a
        a2_v[pl.ds(i * 16, 16)] = a * sq
        sqd_v[pl.ds(i * 16, 16)] = sq
        return carry
    lax.fori_loop(0, NPT // 16, _prep, 0)

    def _gy(ci, carry):
        rbase = node_base + ci * ROWB
        gb = half_base + rbase
        pltpu.sync_copy(h_hbm.at[pl.ds(gb, ROWB)], acc_v)

        def _grp(ng, carry2):
            s16v = dis_v[pl.ds(ci * ROWB + ng * 16, 16)]
            for k in range(16):
                s16 = _lane(s16v, k)
                nn = ng * 16 + k
                for q in range(HALF // 16):
                    y0 = acc_v[nn, pl.ds(q * 16, 16)] * s16
                    acc_v[nn, pl.ds(q * 16, 16)] = y0
                    gbuf_v[nn, pl.ds(q * 16, 16)] = y0 * 0.1
            return carry2
        lax.fori_loop(0, ROWB // 16, _grp, 0)
        pltpu.sync_copy(acc_v, ya_hbm.at[pl.ds(gb, ROWB)])
        pltpu.sync_copy(acc_v, agg_sp.at[pl.ds(rbase, ROWB)])
        pltpu.sync_copy(gbuf_v, g_hbm.at[pl.ds(gb, ROWB)])
        return carry
    lax.fori_loop(0, RB_ITERS, _gy, 0)
    plsc.subcore_barrier()

    ybufs = [ya_hbm, yb_hbm]
    for r in range(K_PROP):
        ycur = ybufs[r % 2]
        final = r == K_PROP - 1
        ynext = res_hbm if final else ybufs[(r + 1) % 2]

        for s in range(NSLOT):
            pltpu.async_copy(ycur.at[asrc_v.at[s]], rows_v[s], sg[s])

        def _edges(jj, carry):
            for s in range(NSLOT):
                cidx = jj * NSLOT + s
                pltpu.make_async_copy(
                    ycur.at[asrc_v.at[cidx]], rows_v[s], sg[s]).wait()
                pltpu.async_copy(
                    rows_v[s], agg_sp.at[adst_v.at[cidx]], ss[s], add=True)
            for s in range(NSLOT):
                cidx = jj * NSLOT + s

                @pl.when(jj < NJ - 1)
                def _():
                    pltpu.make_async_copy(
                        rows_v[s], agg_sp.at[adst_v.at[cidx]], ss[s]).wait()
                    pltpu.async_copy(
                        ycur.at[asrc_v.at[cidx + NSLOT]], rows_v[s], sg[s])
            return carry
        lax.fori_loop(0, NJ, _edges, 0)
        for s in range(NSLOT):
            cidx = (NJ - 1) * NSLOT + s
            pltpu.make_async_copy(
                rows_v[s], agg_sp.at[adst_v.at[cidx]], ss[s]).wait()
        plsc.subcore_barrier()

        av = a2_v if final else a_v

        def _upd(ci, carry):
            rbase = node_base + ci * ROWB
            gb = half_base + rbase
            pltpu.sync_copy(agg_sp.at[pl.ds(rbase, ROWB)], acc_v)
            pltpu.sync_copy(g_hbm.at[pl.ds(gb, ROWB)], gbuf_v)

            def _grp(ng, carry2):
                a16v = av[pl.ds(ci * ROWB + ng * 16, 16)]
                if final:
                    q16v = sqd_v[pl.ds(ci * ROWB + ng * 16, 16)]
                for k in range(16):
                    a16 = _lane(a16v, k)
                    if final:
                        q16 = _lane(q16v, k)
                    nn = ng * 16 + k
                    for q in range(HALF // 16):
                        t = acc_v[nn, pl.ds(q * 16, 16)] * a16
                        gg = gbuf_v[nn, pl.ds(q * 16, 16)]
                        if final:
                            gg = gg * q16
                        acc_v[nn, pl.ds(q * 16, 16)] = t + gg
                return carry2
            lax.fori_loop(0, ROWB // 16, _grp, 0)
            pltpu.sync_copy(acc_v, ynext.at[pl.ds(gb, ROWB)])
            if not final:
                pltpu.sync_copy(acc_v, agg_sp.at[pl.ds(rbase, ROWB)])
            return carry
        lax.fori_loop(0, RB_ITERS, _upd, 0)
        plsc.subcore_barrier()


def _propagate(src_r, dst_r, h_cols):
    mesh = plsc.VectorSubcoreMesh(core_axis_name="c", subcore_axis_name="s")
    f32 = jnp.float32
    out_type = [
        jax.ShapeDtypeStruct((2 * NPAD, HALF), f32),
        jax.ShapeDtypeStruct((2 * NPAD, HALF), f32),
        jax.ShapeDtypeStruct((2 * NPAD, HALF), f32),
        jax.ShapeDtypeStruct((2 * NPAD, HALF), f32),
    ]
    scratch_types = [
        pltpu.VMEM_SHARED((NPAD, HALF), f32),
        pltpu.VMEM_SHARED((NPAD,), f32),
        pltpu.VMEM((NCHUNK, CHUNK), jnp.int32),
        pltpu.VMEM((NCHUNK, CHUNK), jnp.int32),
        pltpu.VMEM((CHUNK,), f32),
        pltpu.VMEM((NPT,), f32),
        pltpu.VMEM((NPT,), f32),
        pltpu.VMEM((NPT,), f32),
        pltpu.VMEM((NPT,), f32),
        pltpu.VMEM((NPT,), f32),
        pltpu.VMEM((CHUNK, HALF), f32),
        pltpu.VMEM((CHUNK, HALF), f32),
        pltpu.VMEM((CHUNK, HALF), f32),
        pltpu.VMEM((CHUNK, HALF), f32),
        pltpu.VMEM((ROWB, HALF), f32),
        pltpu.VMEM((ROWB, HALF), f32),
        pltpu.SemaphoreType.DMA,
        pltpu.SemaphoreType.DMA,
        pltpu.SemaphoreType.DMA,
        pltpu.SemaphoreType.DMA,
        pltpu.SemaphoreType.DMA,
        pltpu.SemaphoreType.DMA,
        pltpu.SemaphoreType.DMA,
        pltpu.SemaphoreType.DMA,
    ]
    res, _, _, _ = pl.kernel(
        _prop_body,
        out_type=out_type,
        mesh=mesh,
        scratch_types=scratch_types,
        compiler_params=pltpu.CompilerParams(use_tc_tiling_on_sc=False),
    )(src_r, dst_r, h_cols)
    return res


def kernel(x, edge_index, W1, b1, Wr, br, W2, b2):
    h = _mlp(x, W1, b1, Wr, br, W2, b2)
    hp = jnp.pad(h, ((0, NPAD - N), (0, 0)))
    h_cols = jnp.concatenate([hp[:, :HALF], hp[:, HALF:]], axis=0)
    padidx = N + (jnp.arange(EPAD - E, dtype=jnp.int32) % (NPAD - N))
    src_r = jnp.concatenate([edge_index[0], padidx]).reshape(TILES, NCHUNK, CHUNK)
    dst_r = jnp.concatenate([edge_index[1], padidx]).reshape(TILES, NCHUNK, CHUNK)
    res = _propagate(src_r, dst_r, h_cols)
    return jnp.concatenate([res[:N], res[NPAD:NPAD + N]], axis=1)


# speedup vs baseline: 21.8167x; 1.0094x over previous
import jax
import jax.numpy as jnp
from jax import lax
from jax.experimental import pallas as pl
from jax.experimental.pallas import tpu as pltpu
from jax.experimental.pallas import tpu_sc as plsc

N = 10000
E = 320000
D = 128
HALF = 64
K_PROP = 10
TILES = 16
NPAD = 10240
NPT = NPAD // TILES
CHUNK = 128
NCHUNK = 160
NSLOT = 5
NJ = NCHUNK // NSLOT
SHIFT = 14
MASK = (1 << SHIFT) - 1
EPAD = TILES * NCHUNK * CHUNK
ROWB = 64
RB_ITERS = NPT // ROWB


def _mlp_body(x_ref, w1_ref, b1_ref, wr_ref, br_ref, w2_ref, b2_ref, o_ref):
    x = x_ref[...]
    h1 = jnp.maximum(
        jnp.dot(x, w1_ref[...], preferred_element_type=jnp.float32) + b1_ref[...], 0.0)
    h2 = jnp.maximum(
        jnp.dot(h1, wr_ref[...], preferred_element_type=jnp.float32) + br_ref[...], 0.0)
    o_ref[...] = jnp.dot(
        h1 + h2, w2_ref[...], preferred_element_type=jnp.float32) + b2_ref[...]


def _mlp(x, W1, b1, Wr, br, W2, b2):
    BM = 2000
    full = lambda i: (0, 0)
    return pl.pallas_call(
        _mlp_body,
        grid=(N // BM,),
        in_specs=[
            pl.BlockSpec((BM, D), lambda i: (i, 0)),
            pl.BlockSpec((D, D), full),
            pl.BlockSpec((1, D), full),
            pl.BlockSpec((D, D), full),
            pl.BlockSpec((1, D), full),
            pl.BlockSpec((D, D), full),
            pl.BlockSpec((1, D), full),
        ],
        out_specs=pl.BlockSpec((BM, D), lambda i: (i, 0)),
        out_shape=jax.ShapeDtypeStruct((N, D), jnp.float32),
    )(x, W1, b1.reshape(1, D), Wr, br.reshape(1, D), W2, b2.reshape(1, D))


def _lane(vec, k):
    idx = jnp.full((16,), k, dtype=jnp.int32)
    return vec.at[idx].get(mode="promise_in_bounds")


def _prop_body(pk_hbm, h_hbm,
               res_hbm, ya_hbm, yb_hbm, g_hbm,
               agg_sp, deg_sp,
               apk_v, idx0_v, idx1_v, idx2_v, idx3_v, idx4_v,
               ones_v, degb_v, dis_v, a_v, a2_v, sqd_v,
               rows_v0, rows_v1, rows_v2, rows_v3, rows_v4, acc_v, gbuf_v,
               sg0, sg1, sg2, sg3, sg4, ss0, ss1, ss2, ss3, ss4):
    rows_v = [rows_v0, rows_v1, rows_v2, rows_v3, rows_v4]
    idx_v = [idx0_v, idx1_v, idx2_v, idx3_v, idx4_v]
    sg = [sg0, sg1, sg2, sg3, sg4]
    ss = [ss0, ss1, ss2, ss3, ss4]
    c = lax.axis_index("c")
    tid = lax.axis_index("s")
    node_base = tid * NPT
    half_base = c * NPAD

    pltpu.sync_copy(pk_hbm.at[tid], apk_v)

    cvec = jnp.full((16,), c * NPAD, dtype=jnp.int32)
    mvec = jnp.full((16,), MASK, dtype=jnp.int32)

    def _decode(cidx, ib):
        def _dec(l, carry):
            p = apk_v[cidx, pl.ds(l * 16, 16)]
            ib[0, pl.ds(l * 16, 16)] = (p & mvec) + cvec
            ib[1, pl.ds(l * 16, 16)] = lax.shift_right_logical(p, SHIFT)
            return carry
        lax.fori_loop(0, CHUNK // 16, _dec, 0)

    ones16 = jnp.ones((16,), jnp.float32)
    zeros16 = jnp.zeros((16,), jnp.float32)
    for l in range(CHUNK // 16):
        ones_v[pl.ds(l * 16, 16)] = ones16

    def _zdeg(i, carry):
        degb_v[pl.ds(i * 16, 16)] = zeros16
        return carry
    lax.fori_loop(0, NPT // 16, _zdeg, 0)
    pltpu.sync_copy(degb_v, deg_sp.at[pl.ds(node_base, NPT)])
    plsc.subcore_barrier()

    def _deg(j, carry):
        _decode(j, idx0_v)
        pltpu.sync_copy(ones_v, deg_sp.at[idx0_v.at[1]], add=True)
        return carry
    lax.fori_loop(0, NCHUNK, _deg, 0)
    plsc.subcore_barrier()

    pltpu.sync_copy(deg_sp.at[pl.ds(node_base, NPT)], degb_v)

    def _prep(i, carry):
        dloc = degb_v[pl.ds(i * 16, 16)] + 1.0
        sq = dloc
        for _ in range(16):
            sq = 0.5 * (sq + dloc / sq)
        r = 1.0 / sq
        a = 0.9 / dloc
        dis_v[pl.ds(i * 16, 16)] = r
        a_v[pl.ds(i * 16, 16)] = a
        a2_v[pl.ds(i * 16, 16)] = a * sq
        sqd_v[pl.ds(i * 16, 16)] = sq
        return carry
    lax.fori_loop(0, NPT // 16, _prep, 0)

    def _gy(ci, carry):
        rbase = node_base + ci * ROWB
        gb = half_base + rbase
        pltpu.sync_copy(h_hbm.at[pl.ds(gb, ROWB)], acc_v)

        def _grp(ng, carry2):
            s16v = dis_v[pl.ds(ci * ROWB + ng * 16, 16)]
            for k in range(16):
                s16 = _lane(s16v, k)
                nn = ng * 16 + k
                for q in range(HALF // 16):
                    y0 = acc_v[nn, pl.ds(q * 16, 16)] * s16
                    acc_v[nn, pl.ds(q * 16, 16)] = y0
                    gbuf_v[nn, pl.ds(q * 16, 16)] = y0 * 0.1
            return carry2
        lax.fori_loop(0, ROWB // 16, _grp, 0)
        pltpu.sync_copy(acc_v, ya_hbm.at[pl.ds(gb, ROWB)])
        pltpu.sync_copy(acc_v, agg_sp.at[pl.ds(rbase, ROWB)])
        pltpu.sync_copy(gbuf_v, g_hbm.at[pl.ds(gb, ROWB)])
        return carry
    lax.fori_loop(0, RB_ITERS, _gy, 0)
    plsc.subcore_barrier()

    ybufs = [ya_hbm, yb_hbm]
    for r in range(K_PROP):
        ycur = ybufs[r % 2]
        final = r == K_PROP - 1
        ynext = res_hbm if final else ybufs[(r + 1) % 2]

        for s in range(NSLOT):
            _decode(s, idx_v[s])
            pltpu.async_copy(ycur.at[idx_v[s].at[0]], rows_v[s], sg[s])

        def _edges(jj, carry):
            for s in range(NSLOT):
                pltpu.make_async_copy(
                    ycur.at[idx_v[s].at[0]], rows_v[s], sg[s]).wait()
                pltpu.async_copy(
                    rows_v[s], agg_sp.at[idx_v[s].at[1]], ss[s], add=True)
            for s in range(NSLOT):
                cidx = jj * NSLOT + s

                @pl.when(jj < NJ - 1)
                def _():
                    pltpu.make_async_copy(
                        rows_v[s], agg_sp.at[idx_v[s].at[1]], ss[s]).wait()
                    _decode(cidx + NSLOT, idx_v[s])
                    pltpu.async_copy(
                        ycur.at[idx_v[s].at[0]], rows_v[s], sg[s])
            return carry
        lax.fori_loop(0, NJ, _edges, 0)
        for s in range(NSLOT):
            pltpu.make_async_copy(
                rows_v[s], agg_sp.at[idx_v[s].at[1]], ss[s]).wait()
        plsc.subcore_barrier()

        av = a2_v if final else a_v

        def _upd(ci, carry):
            rbase = node_base + ci * ROWB
            gb = half_base + rbase
            pltpu.sync_copy(agg_sp.at[pl.ds(rbase, ROWB)], acc_v)
            pltpu.sync_copy(g_hbm.at[pl.ds(gb, ROWB)], gbuf_v)

            def _grp(ng, carry2):
                a16v = av[pl.ds(ci * ROWB + ng * 16, 16)]
                if final:
                    q16v = sqd_v[pl.ds(ci * ROWB + ng * 16, 16)]
                for k in range(16):
                    a16 = _lane(a16v, k)
                    if final:
                        q16 = _lane(q16v, k)
                    nn = ng * 16 + k
                    for q in range(HALF // 16):
                        t = acc_v[nn, pl.ds(q * 16, 16)] * a16
                        gg = gbuf_v[nn, pl.ds(q * 16, 16)]
                        if final:
                            gg = gg * q16
                        acc_v[nn, pl.ds(q * 16, 16)] = t + gg
                return carry2
            lax.fori_loop(0, ROWB // 16, _grp, 0)
            pltpu.sync_copy(acc_v, ynext.at[pl.ds(gb, ROWB)])
            if not final:
                pltpu.sync_copy(acc_v, agg_sp.at[pl.ds(rbase, ROWB)])
            return carry
        lax.fori_loop(0, RB_ITERS, _upd, 0)
        plsc.subcore_barrier()


def _propagate(pk_r, h_cols):
    mesh = plsc.VectorSubcoreMesh(core_axis_name="c", subcore_axis_name="s")
    f32 = jnp.float32
    out_type = [
        jax.ShapeDtypeStruct((2 * NPAD, HALF), f32),
        jax.ShapeDtypeStruct((2 * NPAD, HALF), f32),
        jax.ShapeDtypeStruct((2 * NPAD, HALF), f32),
        jax.ShapeDtypeStruct((2 * NPAD, HALF), f32),
    ]
    scratch_types = (
        [
            pltpu.VMEM_SHARED((NPAD, HALF), f32),
            pltpu.VMEM_SHARED((NPAD,), f32),
            pltpu.VMEM((NCHUNK, CHUNK), jnp.int32),
        ]
        + [pltpu.VMEM((2, CHUNK), jnp.int32)] * NSLOT
        + [
            pltpu.VMEM((CHUNK,), f32),
            pltpu.VMEM((NPT,), f32),
            pltpu.VMEM((NPT,), f32),
            pltpu.VMEM((NPT,), f32),
            pltpu.VMEM((NPT,), f32),
            pltpu.VMEM((NPT,), f32),
        ]
        + [pltpu.VMEM((CHUNK, HALF), f32)] * NSLOT
        + [
            pltpu.VMEM((ROWB, HALF), f32),
            pltpu.VMEM((ROWB, HALF), f32),
        ]
        + [pltpu.SemaphoreType.DMA] * (2 * NSLOT)
    )
    res, _, _, _ = pl.kernel(
        _prop_body,
        out_type=out_type,
        mesh=mesh,
        scratch_types=scratch_types,
        compiler_params=pltpu.CompilerParams(use_tc_tiling_on_sc=False),
    )(pk_r, h_cols)
    return res


def kernel(x, edge_index, W1, b1, Wr, br, W2, b2):
    h = _mlp(x, W1, b1, Wr, br, W2, b2)
    hp = jnp.pad(h, ((0, NPAD - N), (0, 0)))
    h_cols = jnp.concatenate([hp[:, :HALF], hp[:, HALF:]], axis=0)
    padidx = N + (jnp.arange(EPAD - E, dtype=jnp.int32) % (NPAD - N))
    src_f = jnp.concatenate([edge_index[0], padidx])
    dst_f = jnp.concatenate([edge_index[1], padidx])
    pk_r = (src_f | (dst_f << SHIFT)).reshape(TILES, NCHUNK, CHUNK)
    res = _propagate(pk_r, h_cols)
    return jnp.concatenate([res[:N], res[NPAD:NPAD + N]], axis=1)


# speedup vs baseline: 22.0506x; 1.0107x over previous
import jax
import jax.numpy as jnp
from jax import lax
from jax.experimental import pallas as pl
from jax.experimental.pallas import tpu as pltpu
from jax.experimental.pallas import tpu_sc as plsc

N = 10000
E = 320000
D = 128
HALF = 64
K_PROP = 10
TILES = 16
NPAD = 10240
NPT = NPAD // TILES
CHUNK = 128
NCHUNK = 160
NSLOT = 5
NJ = NCHUNK // NSLOT
SHIFT = 14
MASK = (1 << SHIFT) - 1
EPAD = TILES * NCHUNK * CHUNK
ROWB = 64
RB_ITERS = NPT // ROWB


def _mlp_body(x_ref, w1_ref, b1_ref, wr_ref, br_ref, w2_ref, b2_ref, o_ref):
    x = x_ref[...]
    h1 = jnp.maximum(
        jnp.dot(x, w1_ref[...], preferred_element_type=jnp.float32) + b1_ref[...], 0.0)
    h2 = jnp.maximum(
        jnp.dot(h1, wr_ref[...], preferred_element_type=jnp.float32) + br_ref[...], 0.0)
    o_ref[...] = jnp.dot(
        h1 + h2, w2_ref[...], preferred_element_type=jnp.float32) + b2_ref[...]


def _mlp(x, W1, b1, Wr, br, W2, b2):
    BM = 2000
    full = lambda i: (0, 0)
    return pl.pallas_call(
        _mlp_body,
        grid=(N // BM,),
        in_specs=[
            pl.BlockSpec((BM, D), lambda i: (i, 0)),
            pl.BlockSpec((D, D), full),
            pl.BlockSpec((1, D), full),
            pl.BlockSpec((D, D), full),
            pl.BlockSpec((1, D), full),
            pl.BlockSpec((D, D), full),
            pl.BlockSpec((1, D), full),
        ],
        out_specs=pl.BlockSpec((BM, D), lambda i: (i, 0)),
        out_shape=jax.ShapeDtypeStruct((N, D), jnp.float32),
    )(x, W1, b1.reshape(1, D), Wr, br.reshape(1, D), W2, b2.reshape(1, D))


def _lane(vec, k):
    idx = jnp.full((16,), k, dtype=jnp.int32)
    return vec.at[idx].get(mode="promise_in_bounds")


def _prop_body(pk_hbm, h_hbm,
               res_hbm, ya_hbm, yb_hbm, g_hbm,
               agg_sp, deg_sp,
               apk_v, idx0_v, idx1_v, idx2_v, idx3_v, idx4_v,
               ones_v, degb_v, dis_v, a_v, a2_v, sqd_v,
               rows_v0, rows_v1, rows_v2, rows_v3, rows_v4, acc_v, gbuf_v,
               sg0, sg1, sg2, sg3, sg4, ss0, ss1, ss2, ss3, ss4):
    rows_v = [rows_v0, rows_v1, rows_v2, rows_v3, rows_v4]
    idx_v = [idx0_v, idx1_v, idx2_v, idx3_v, idx4_v]
    sg = [sg0, sg1, sg2, sg3, sg4]
    ss = [ss0, ss1, ss2, ss3, ss4]
    c = lax.axis_index("c")
    tid = lax.axis_index("s")
    node_base = tid * NPT
    half_base = c * NPAD

    pltpu.sync_copy(pk_hbm.at[tid], apk_v)

    cvec = jnp.full((16,), c * NPAD, dtype=jnp.int32)
    mvec = jnp.full((16,), MASK, dtype=jnp.int32)

    def _decode(cidx, ib):
        def _dec(l, carry):
            p = apk_v[cidx, pl.ds(l * 16, 16)]
            ib[0, pl.ds(l * 16, 16)] = (p & mvec) + cvec
            ib[1, pl.ds(l * 16, 16)] = lax.shift_right_logical(p, SHIFT)
            return carry
        lax.fori_loop(0, CHUNK // 16, _dec, 0)

    ones16 = jnp.ones((16,), jnp.float32)
    zeros16 = jnp.zeros((16,), jnp.float32)
    for l in range(CHUNK // 16):
        ones_v[pl.ds(l * 16, 16)] = ones16

    def _zdeg(i, carry):
        degb_v[pl.ds(i * 16, 16)] = zeros16
        return carry
    lax.fori_loop(0, NPT // 16, _zdeg, 0)
    pltpu.sync_copy(degb_v, deg_sp.at[pl.ds(node_base, NPT)])
    plsc.subcore_barrier()

    for s in range(NSLOT):
        _decode(s, idx_v[s])
        pltpu.async_copy(ones_v, deg_sp.at[idx_v[s].at[1]], ss[s], add=True)

    def _deg(jj, carry):
        for s in range(NSLOT):
            cidx = jj * NSLOT + s

            @pl.when(jj < NJ - 1)
            def _():
                pltpu.make_async_copy(
                    ones_v, deg_sp.at[idx_v[s].at[1]], ss[s]).wait()
                _decode(cidx + NSLOT, idx_v[s])
                pltpu.async_copy(
                    ones_v, deg_sp.at[idx_v[s].at[1]], ss[s], add=True)
        return carry
    lax.fori_loop(0, NJ, _deg, 0)
    for s in range(NSLOT):
        pltpu.make_async_copy(ones_v, deg_sp.at[idx_v[s].at[1]], ss[s]).wait()
    plsc.subcore_barrier()

    pltpu.sync_copy(deg_sp.at[pl.ds(node_base, NPT)], degb_v)

    def _prep(i, carry):
        dloc = degb_v[pl.ds(i * 16, 16)] + 1.0
        sq = dloc
        for _ in range(16):
            sq = 0.5 * (sq + dloc / sq)
        r = 1.0 / sq
        a = 0.9 / dloc
        dis_v[pl.ds(i * 16, 16)] = r
        a_v[pl.ds(i * 16, 16)] = a
        a2_v[pl.ds(i * 16, 16)] = a * sq
        sqd_v[pl.ds(i * 16, 16)] = sq
        return carry
    lax.fori_loop(0, NPT // 16, _prep, 0)

    def _gy(ci, carry):
        rbase = node_base + ci * ROWB
        gb = half_base + rbase
        pltpu.sync_copy(h_hbm.at[pl.ds(gb, ROWB)], acc_v)

        def _grp(ng, carry2):
            s16v = dis_v[pl.ds(ci * ROWB + ng * 16, 16)]
            for k in range(16):
                s16 = _lane(s16v, k)
                nn = ng * 16 + k
                for q in range(HALF // 16):
                    y0 = acc_v[nn, pl.ds(q * 16, 16)] * s16
                    acc_v[nn, pl.ds(q * 16, 16)] = y0
                    gbuf_v[nn, pl.ds(q * 16, 16)] = y0 * 0.1
            return carry2
        lax.fori_loop(0, ROWB // 16, _grp, 0)
        pltpu.sync_copy(acc_v, ya_hbm.at[pl.ds(gb, ROWB)])
        pltpu.sync_copy(acc_v, agg_sp.at[pl.ds(rbase, ROWB)])
        pltpu.sync_copy(gbuf_v, g_hbm.at[pl.ds(gb, ROWB)])
        return carry
    lax.fori_loop(0, RB_ITERS, _gy, 0)
    plsc.subcore_barrier()

    ybufs = [ya_hbm, yb_hbm]
    for r in range(K_PROP):
        ycur = ybufs[r % 2]
        final = r == K_PROP - 1
        ynext = res_hbm if final else ybufs[(r + 1) % 2]

        for s in range(NSLOT):
            _decode(s, idx_v[s])
            pltpu.async_copy(ycur.at[idx_v[s].at[0]], rows_v[s], sg[s])

        def _edges(jj, carry):
            for s in range(NSLOT):
                pltpu.make_async_copy(
                    ycur.at[idx_v[s].at[0]], rows_v[s], sg[s]).wait()
                pltpu.async_copy(
                    rows_v[s], agg_sp.at[idx_v[s].at[1]], ss[s], add=True)
            for s in range(NSLOT):
                cidx = jj * NSLOT + s

                @pl.when(jj < NJ - 1)
                def _():
                    pltpu.make_async_copy(
                        rows_v[s], agg_sp.at[idx_v[s].at[1]], ss[s]).wait()
                    _decode(cidx + NSLOT, idx_v[s])
                    pltpu.async_copy(
                        ycur.at[idx_v[s].at[0]], rows_v[s], sg[s])
            return carry
        lax.fori_loop(0, NJ, _edges, 0)
        for s in range(NSLOT):
            pltpu.make_async_copy(
                rows_v[s], agg_sp.at[idx_v[s].at[1]], ss[s]).wait()
        plsc.subcore_barrier()

        av = a2_v if final else a_v

        def _upd(ci, carry):
            rbase = node_base + ci * ROWB
            gb = half_base + rbase
            pltpu.sync_copy(agg_sp.at[pl.ds(rbase, ROWB)], acc_v)
            pltpu.sync_copy(g_hbm.at[pl.ds(gb, ROWB)], gbuf_v)

            def _grp(ng, carry2):
                a16v = av[pl.ds(ci * ROWB + ng * 16, 16)]
                if final:
                    q16v = sqd_v[pl.ds(ci * ROWB + ng * 16, 16)]
                for k in range(16):
                    a16 = _lane(a16v, k)
                    if final:
                        q16 = _lane(q16v, k)
                    nn = ng * 16 + k
                    for q in range(HALF // 16):
                        t = acc_v[nn, pl.ds(q * 16, 16)] * a16
                        gg = gbuf_v[nn, pl.ds(q * 16, 16)]
                        if final:
                            gg = gg * q16
                        acc_v[nn, pl.ds(q * 16, 16)] = t + gg
                return carry2
            lax.fori_loop(0, ROWB // 16, _grp, 0)
            pltpu.sync_copy(acc_v, ynext.at[pl.ds(gb, ROWB)])
            if not final:
                pltpu.sync_copy(acc_v, agg_sp.at[pl.ds(rbase, ROWB)])
            return carry
        lax.fori_loop(0, RB_ITERS, _upd, 0)
        plsc.subcore_barrier()


def _propagate(pk_r, h_cols):
    mesh = plsc.VectorSubcoreMesh(core_axis_name="c", subcore_axis_name="s")
    f32 = jnp.float32
    out_type = [
        jax.ShapeDtypeStruct((2 * NPAD, HALF), f32),
        jax.ShapeDtypeStruct((2 * NPAD, HALF), f32),
        jax.ShapeDtypeStruct((2 * NPAD, HALF), f32),
        jax.ShapeDtypeStruct((2 * NPAD, HALF), f32),
    ]
    scratch_types = (
        [
            pltpu.VMEM_SHARED((NPAD, HALF), f32),
            pltpu.VMEM_SHARED((NPAD,), f32),
            pltpu.VMEM((NCHUNK, CHUNK), jnp.int32),
        ]
        + [pltpu.VMEM((2, CHUNK), jnp.int32)] * NSLOT
        + [
            pltpu.VMEM((CHUNK,), f32),
            pltpu.VMEM((NPT,), f32),
            pltpu.VMEM((NPT,), f32),
            pltpu.VMEM((NPT,), f32),
            pltpu.VMEM((NPT,), f32),
            pltpu.VMEM((NPT,), f32),
        ]
        + [pltpu.VMEM((CHUNK, HALF), f32)] * NSLOT
        + [
            pltpu.VMEM((ROWB, HALF), f32),
            pltpu.VMEM((ROWB, HALF), f32),
        ]
        + [pltpu.SemaphoreType.DMA] * (2 * NSLOT)
    )
    res, _, _, _ = pl.kernel(
        _prop_body,
        out_type=out_type,
        mesh=mesh,
        scratch_types=scratch_types,
        compiler_params=pltpu.CompilerParams(use_tc_tiling_on_sc=False),
    )(pk_r, h_cols)
    return res


def kernel(x, edge_index, W1, b1, Wr, br, W2, b2):
    h = _mlp(x, W1, b1, Wr, br, W2, b2)
    hp = jnp.pad(h, ((0, NPAD - N), (0, 0)))
    h_cols = jnp.concatenate([hp[:, :HALF], hp[:, HALF:]], axis=0)
    padidx = N + (jnp.arange(EPAD - E, dtype=jnp.int32) % (NPAD - N))
    src_f = jnp.concatenate([edge_index[0], padidx])
    dst_f = jnp.concatenate([edge_index[1], padidx])
    pk_r = (src_f | (dst_f << SHIFT)).reshape(TILES, NCHUNK, CHUNK)
    res = _propagate(pk_r, h_cols)
    return jnp.concatenate([res[:N], res[NPAD:NPAD + N]], axis=1)


# speedup vs baseline: 22.6963x; 1.0293x over previous
import jax
import jax.numpy as jnp
from jax import lax
from jax.experimental import pallas as pl
from jax.experimental.pallas import tpu as pltpu
from jax.experimental.pallas import tpu_sc as plsc

N = 10000
E = 320000
D = 128
HALF = 64
K_PROP = 10
TILES = 16
NPAD = 10240
NPT = NPAD // TILES
CHUNK = 128
NCHUNK = 160
NSLOT = 5
NJ = NCHUNK // NSLOT
SHIFT = 14
MASK = (1 << SHIFT) - 1
EPAD = TILES * NCHUNK * CHUNK
ROWB = 64
RB_ITERS = NPT // ROWB


def _mlp_body(x_ref, w1_ref, b1_ref, wr_ref, br_ref, w2_ref, b2_ref, o_ref):
    x = x_ref[...]
    h1 = jnp.maximum(
        jnp.dot(x, w1_ref[...], preferred_element_type=jnp.float32) + b1_ref[...], 0.0)
    h2 = jnp.maximum(
        jnp.dot(h1, wr_ref[...], preferred_element_type=jnp.float32) + br_ref[...], 0.0)
    o_ref[...] = jnp.dot(
        h1 + h2, w2_ref[...], preferred_element_type=jnp.float32) + b2_ref[...]


def _mlp(x, W1, b1, Wr, br, W2, b2):
    BM = 2000
    full = lambda i: (0, 0)
    return pl.pallas_call(
        _mlp_body,
        grid=(N // BM,),
        in_specs=[
            pl.BlockSpec((BM, D), lambda i: (i, 0)),
            pl.BlockSpec((D, D), full),
            pl.BlockSpec((1, D), full),
            pl.BlockSpec((D, D), full),
            pl.BlockSpec((1, D), full),
            pl.BlockSpec((D, D), full),
            pl.BlockSpec((1, D), full),
        ],
        out_specs=pl.BlockSpec((BM, D), lambda i: (i, 0)),
        out_shape=jax.ShapeDtypeStruct((N, D), jnp.float32),
    )(x, W1, b1.reshape(1, D), Wr, br.reshape(1, D), W2, b2.reshape(1, D))


def _lane(vec, k):
    idx = jnp.full((16,), k, dtype=jnp.int32)
    return vec.at[idx].get(mode="promise_in_bounds")


def _prop_body(pk_hbm, h_hbm,
               res_hbm, ya_hbm, yb_hbm, g_hbm,
               agg_sp, deg_sp,
               apk_v, idx0_v, idx1_v, idx2_v, idx3_v, idx4_v,
               ones_v, degb_v, dis_v, a_v, a2_v, sqd_v,
               rows_v0, rows_v1, rows_v2, rows_v3, rows_v4, acc_v, gbuf_v,
               sg0, sg1, sg2, sg3, sg4, ss0, ss1, ss2, ss3, ss4):
    rows_v = [rows_v0, rows_v1, rows_v2, rows_v3, rows_v4]
    idx_v = [idx0_v, idx1_v, idx2_v, idx3_v, idx4_v]
    sg = [sg0, sg1, sg2, sg3, sg4]
    ss = [ss0, ss1, ss2, ss3, ss4]
    c = lax.axis_index("c")
    tid = lax.axis_index("s")
    node_base = tid * NPT
    half_base = c * NPAD

    pltpu.sync_copy(pk_hbm.at[tid], apk_v)

    cvec = jnp.full((16,), c * NPAD, dtype=jnp.int32)
    mvec = jnp.full((16,), MASK, dtype=jnp.int32)

    def _decode(cidx, ib):
        def _dec(l, carry):
            p = apk_v[cidx, pl.ds(l * 16, 16)]
            ib[0, pl.ds(l * 16, 16)] = (p & mvec) + cvec
            ib[1, pl.ds(l * 16, 16)] = lax.shift_right_logical(p, SHIFT)
            return carry
        lax.fori_loop(0, CHUNK // 16, _dec, 0)

    ones16 = jnp.ones((16,), jnp.float32)
    zeros16 = jnp.zeros((16,), jnp.float32)
    for l in range(CHUNK // 16):
        ones_v[pl.ds(l * 16, 16)] = ones16

    def _zdeg(i, carry):
        degb_v[pl.ds(i * 16, 16)] = zeros16
        return carry
    lax.fori_loop(0, NPT // 16, _zdeg, 0)
    pltpu.sync_copy(degb_v, deg_sp.at[pl.ds(node_base, NPT)])
    plsc.subcore_barrier()

    for s in range(NSLOT):
        _decode(s, idx_v[s])
        pltpu.async_copy(ones_v, deg_sp.at[idx_v[s].at[1]], ss[s], add=True)

    def _deg(jj, carry):
        for s in range(NSLOT):
            cidx = jj * NSLOT + s

            @pl.when(jj < NJ - 1)
            def _():
                pltpu.make_async_copy(
                    ones_v, deg_sp.at[idx_v[s].at[1]], ss[s]).wait()
                _decode(cidx + NSLOT, idx_v[s])
                pltpu.async_copy(
                    ones_v, deg_sp.at[idx_v[s].at[1]], ss[s], add=True)
        return carry
    lax.fori_loop(0, NJ, _deg, 0)
    for s in range(NSLOT):
        pltpu.make_async_copy(ones_v, deg_sp.at[idx_v[s].at[1]], ss[s]).wait()
    plsc.subcore_barrier()

    pltpu.sync_copy(deg_sp.at[pl.ds(node_base, NPT)], degb_v)

    def _prep(i, carry):
        dloc = degb_v[pl.ds(i * 16, 16)] + 1.0
        sq = dloc
        for _ in range(16):
            sq = 0.5 * (sq + dloc / sq)
        r = 1.0 / sq
        a = 0.9 / dloc
        dis_v[pl.ds(i * 16, 16)] = r
        a_v[pl.ds(i * 16, 16)] = a
        a2_v[pl.ds(i * 16, 16)] = a * sq
        sqd_v[pl.ds(i * 16, 16)] = sq
        return carry
    lax.fori_loop(0, NPT // 16, _prep, 0)

    def _gy(ci, carry):
        rbase = node_base + ci * ROWB
        gb = half_base + rbase
        pltpu.sync_copy(h_hbm.at[pl.ds(gb, ROWB)], acc_v)

        def _grp(ng, carry2):
            s16v = dis_v[pl.ds(ci * ROWB + ng * 16, 16)]
            for k in range(16):
                s16 = _lane(s16v, k)
                nn = ng * 16 + k
                for q in range(HALF // 16):
                    y0 = acc_v[nn, pl.ds(q * 16, 16)] * s16
                    acc_v[nn, pl.ds(q * 16, 16)] = y0
                    gbuf_v[nn, pl.ds(q * 16, 16)] = y0 * 0.1
            return carry2
        lax.fori_loop(0, ROWB // 16, _grp, 0)
        pltpu.sync_copy(acc_v, ya_hbm.at[pl.ds(gb, ROWB)])
        pltpu.sync_copy(acc_v, agg_sp.at[pl.ds(rbase, ROWB)])
        pltpu.sync_copy(gbuf_v, g_hbm.at[pl.ds(gb, ROWB)])
        return carry
    lax.fori_loop(0, RB_ITERS, _gy, 0)
    plsc.subcore_barrier()

    ybufs = [ya_hbm, yb_hbm]
    for r in range(K_PROP):
        ycur = ybufs[r % 2]
        final = r == K_PROP - 1
        ynext = res_hbm if final else ybufs[(r + 1) % 2]

        for s in range(NSLOT):
            _decode(s, idx_v[s])
            pltpu.async_copy(ycur.at[idx_v[s].at[0]], rows_v[s], sg[s])

        def _edges(jj, carry):
            for s in range(NSLOT):
                pltpu.make_async_copy(
                    ycur.at[idx_v[s].at[0]], rows_v[s], sg[s]).wait()
                pltpu.async_copy(
                    rows_v[s], agg_sp.at[idx_v[s].at[1]], ss[s], add=True)
            for s in range(NSLOT):
                cidx = jj * NSLOT + s

                @pl.when(jj < NJ - 1)
                def _():
                    pltpu.make_async_copy(
                        rows_v[s], agg_sp.at[idx_v[s].at[1]], ss[s]).wait()
                    _decode(cidx + NSLOT, idx_v[s])
                    pltpu.async_copy(
                        ycur.at[idx_v[s].at[0]], rows_v[s], sg[s])
            return carry
        lax.fori_loop(0, NJ, _edges, 0)
        for s in range(NSLOT):
            pltpu.make_async_copy(
                rows_v[s], agg_sp.at[idx_v[s].at[1]], ss[s]).wait()
        plsc.subcore_barrier()

        av = a2_v if final else a_v

        def _upd(ci, carry):
            rbase = node_base + ci * ROWB
            gb = half_base + rbase

            @pl.when(ci > 0)
            def _():
                pgb = gb - ROWB
                prb = rbase - ROWB
                if final:
                    pltpu.make_async_copy(
                        acc_v, res_hbm.at[pl.ds(pgb, ROWB)], ss0).wait()
                else:
                    pltpu.make_async_copy(
                        acc_v, ynext.at[pl.ds(pgb, ROWB)], ss0).wait()
                    pltpu.make_async_copy(
                        acc_v, agg_sp.at[pl.ds(prb, ROWB)], ss1).wait()

            pltpu.async_copy(agg_sp.at[pl.ds(rbase, ROWB)], acc_v, sg0)
            pltpu.async_copy(g_hbm.at[pl.ds(gb, ROWB)], gbuf_v, sg1)
            pltpu.make_async_copy(agg_sp.at[pl.ds(rbase, ROWB)], acc_v, sg0).wait()
            pltpu.make_async_copy(g_hbm.at[pl.ds(gb, ROWB)], gbuf_v, sg1).wait()

            def _grp(ng, carry2):
                a16v = av[pl.ds(ci * ROWB + ng * 16, 16)]
                if final:
                    q16v = sqd_v[pl.ds(ci * ROWB + ng * 16, 16)]
                for k in range(16):
                    a16 = _lane(a16v, k)
                    if final:
                        q16 = _lane(q16v, k)
                    nn = ng * 16 + k
                    for q in range(HALF // 16):
                        t = acc_v[nn, pl.ds(q * 16, 16)] * a16
                        gg = gbuf_v[nn, pl.ds(q * 16, 16)]
                        if final:
                            gg = gg * q16
                        acc_v[nn, pl.ds(q * 16, 16)] = t + gg
                return carry2
            lax.fori_loop(0, ROWB // 16, _grp, 0)
            if final:
                pltpu.async_copy(acc_v, res_hbm.at[pl.ds(gb, ROWB)], ss0)
            else:
                pltpu.async_copy(acc_v, ynext.at[pl.ds(gb, ROWB)], ss0)
                pltpu.async_copy(acc_v, agg_sp.at[pl.ds(rbase, ROWB)], ss1)
            return carry
        lax.fori_loop(0, RB_ITERS, _upd, 0)
        lgb = half_base + node_base + (RB_ITERS - 1) * ROWB
        lrb = node_base + (RB_ITERS - 1) * ROWB
        if final:
            pltpu.make_async_copy(acc_v, res_hbm.at[pl.ds(lgb, ROWB)], ss0).wait()
        else:
            pltpu.make_async_copy(acc_v, ynext.at[pl.ds(lgb, ROWB)], ss0).wait()
            pltpu.make_async_copy(acc_v, agg_sp.at[pl.ds(lrb, ROWB)], ss1).wait()
        plsc.subcore_barrier()


def _propagate(pk_r, h_cols):
    mesh = plsc.VectorSubcoreMesh(core_axis_name="c", subcore_axis_name="s")
    f32 = jnp.float32
    out_type = [
        jax.ShapeDtypeStruct((2 * NPAD, HALF), f32),
        jax.ShapeDtypeStruct((2 * NPAD, HALF), f32),
        jax.ShapeDtypeStruct((2 * NPAD, HALF), f32),
        jax.ShapeDtypeStruct((2 * NPAD, HALF), f32),
    ]
    scratch_types = (
        [
            pltpu.VMEM_SHARED((NPAD, HALF), f32),
            pltpu.VMEM_SHARED((NPAD,), f32),
            pltpu.VMEM((NCHUNK, CHUNK), jnp.int32),
        ]
        + [pltpu.VMEM((2, CHUNK), jnp.int32)] * NSLOT
        + [
            pltpu.VMEM((CHUNK,), f32),
            pltpu.VMEM((NPT,), f32),
            pltpu.VMEM((NPT,), f32),
            pltpu.VMEM((NPT,), f32),
            pltpu.VMEM((NPT,), f32),
            pltpu.VMEM((NPT,), f32),
        ]
        + [pltpu.VMEM((CHUNK, HALF), f32)] * NSLOT
        + [
            pltpu.VMEM((ROWB, HALF), f32),
            pltpu.VMEM((ROWB, HALF), f32),
        ]
        + [pltpu.SemaphoreType.DMA] * (2 * NSLOT)
    )
    res, _, _, _ = pl.kernel(
        _prop_body,
        out_type=out_type,
        mesh=mesh,
        scratch_types=scratch_types,
        compiler_params=pltpu.CompilerParams(use_tc_tiling_on_sc=False),
    )(pk_r, h_cols)
    return res


def kernel(x, edge_index, W1, b1, Wr, br, W2, b2):
    h = _mlp(x, W1, b1, Wr, br, W2, b2)
    hp = jnp.pad(h, ((0, NPAD - N), (0, 0)))
    h_cols = jnp.concatenate([hp[:, :HALF], hp[:, HALF:]], axis=0)
    padidx = N + (jnp.arange(EPAD - E, dtype=jnp.int32) % (NPAD - N))
    src_f = jnp.concatenate([edge_index[0], padidx])
    dst_f = jnp.concatenate([edge_index[1], padidx])
    pk_r = (src_f | (dst_f << SHIFT)).reshape(TILES, NCHUNK, CHUNK)
    res = _propagate(pk_r, h_cols)
    return jnp.concatenate([res[:N], res[NPAD:NPAD + N]], axis=1)
